# baseline (speedup 1.0000x reference)
import functools

import jax
import jax.numpy as jnp
from jax.experimental import pallas as pl


def _norm_body(x_ref, o_ref, *, block_rows, total_wide_rows, d):
    i = pl.program_id(0)
    x = x_ref[...]
    lft = x[:, :d]
    rgt = x[:, d:]
    inv_l = jax.lax.rsqrt(jnp.sum(lft * lft, axis=1, keepdims=True))
    inv_r = jax.lax.rsqrt(jnp.sum(rgt * rgt, axis=1, keepdims=True))
    wrow = i * block_rows + jax.lax.broadcasted_iota(
        jnp.int32, (block_rows, 1), 0)
    inv_r = jnp.where(wrow == total_wide_rows - 1, 1.0, inv_r)
    o_ref[...] = jnp.concatenate([lft * inv_l, rgt * inv_r], axis=1)


def kernel(entity_embds, rel_embds):
    n, d = entity_embds.shape
    wide = entity_embds.reshape(n // 2, 2 * d)
    block = 10000
    out = pl.pallas_call(
        functools.partial(_norm_body, block_rows=block,
                          total_wide_rows=n // 2, d=d),
        grid=((n // 2) // block,),
        in_specs=[pl.BlockSpec((block, 2 * d), lambda i: (i, 0))],
        out_specs=pl.BlockSpec((block, 2 * d), lambda i: (i, 0)),
        out_shape=jax.ShapeDtypeStruct((n // 2, 2 * d), entity_embds.dtype),
    )(wide)
    return (out.reshape(n, d), rel_embds)


# speedup vs baseline: 1.4314x; 1.4314x over previous
import functools

import jax
import jax.numpy as jnp
from jax.experimental import pallas as pl


def _norm_body(x_ref, o_ref, *, block_rows, total_rows):
    i = pl.program_id(0)
    x = x_ref[...]
    ssq = jnp.sum(x * x, axis=1, keepdims=True)
    inv = jax.lax.rsqrt(ssq)
    row = i * block_rows + jax.lax.broadcasted_iota(jnp.int32, (block_rows, 1), 0)
    scale = jnp.where(row == total_rows - 1, 1.0, inv)
    o_ref[...] = x * scale


def kernel(entity_embds, rel_embds):
    n, d = entity_embds.shape
    block = 20000
    out = pl.pallas_call(
        functools.partial(_norm_body, block_rows=block, total_rows=n),
        grid=(n // block,),
        in_specs=[pl.BlockSpec((block, d), lambda i: (i, 0))],
        out_specs=pl.BlockSpec((block, d), lambda i: (i, 0)),
        out_shape=jax.ShapeDtypeStruct((n, d), entity_embds.dtype),
    )(entity_embds)
    return (out, rel_embds)
